# Initial kernel scaffold; baseline (speedup 1.0000x reference)
import functools

import jax
import jax.numpy as jnp
from jax import lax
from jax.experimental import pallas as pl
from jax.experimental.pallas import tpu as pltpu
from jax.experimental.pallas import tpu_sc as plsc

_N = 10000
_E = 320000
_H = 128
_NORM_INV = 0.01
_EPS_LN = 1e-5
_EPS_R = 1e-8

_NC = 2
_NS = 16
_NW = _NC * _NS
_C = 128
_NCH = _E // _C
_BASE_CH = _NCH // _NW
_EXTRA = _NCH - _BASE_CH * _NW
_RPT = _N // _NS


def _mesh():
    return plsc.VectorSubcoreMesh(core_axis_name="c", subcore_axis_name="s")


def _worker_id():
    return lax.axis_index("s") * _NC + lax.axis_index("c")


def _g4_body(h_hbm, cp_hbm, row_hbm, col_hbm,
             src_o, tgt_o, cr_o, cc_o,
             idx_r, idx_c, bs, bt, bcr, bcc, sem):
    w = _worker_id()

    def do_chunk(ci):
        base = ci * _C
        pltpu.sync_copy(row_hbm.at[pl.ds(base, _C)], idx_r)
        pltpu.sync_copy(col_hbm.at[pl.ds(base, _C)], idx_c)
        c1 = pltpu.async_copy(h_hbm.at[idx_r], bs, sem)
        c2 = pltpu.async_copy(h_hbm.at[idx_c], bt, sem)
        c3 = pltpu.async_copy(cp_hbm.at[idx_r], bcr, sem)
        c4 = pltpu.async_copy(cp_hbm.at[idx_c], bcc, sem)
        c1.wait()
        c2.wait()
        c3.wait()
        c4.wait()
        pltpu.sync_copy(bs, src_o.at[pl.ds(base, _C)])
        pltpu.sync_copy(bt, tgt_o.at[pl.ds(base, _C)])
        pltpu.sync_copy(bcr, cr_o.at[pl.ds(base, _C)])
        pltpu.sync_copy(bcc, cc_o.at[pl.ds(base, _C)])

    def body(k, carry):
        do_chunk(w + k * _NW)
        return carry

    lax.fori_loop(0, _BASE_CH, body, 0)

    @pl.when(w < _EXTRA)
    def _():
        do_chunk(_BASE_CH * _NW + w)


def _gather4(h, cpad, row, col):
    f = functools.partial(
        pl.kernel, mesh=_mesh(),
        out_type=[
            jax.ShapeDtypeStruct((_E, _H), jnp.float32),
            jax.ShapeDtypeStruct((_E, _H), jnp.float32),
            jax.ShapeDtypeStruct((_E, 8), jnp.float32),
            jax.ShapeDtypeStruct((_E, 8), jnp.float32),
        ],
        scratch_types=[
            pltpu.VMEM((_C,), jnp.int32),
            pltpu.VMEM((_C,), jnp.int32),
            pltpu.VMEM((_C, _H), jnp.float32),
            pltpu.VMEM((_C, _H), jnp.float32),
            pltpu.VMEM((_C, 8), jnp.float32),
            pltpu.VMEM((_C, 8), jnp.float32),
            pltpu.SemaphoreType.DMA,
        ],
    )(_g4_body)
    return f(h, cpad, row, col)


def _g2_body(h_hbm, row_hbm, col_hbm, src_o, tgt_o,
             idx_r, idx_c, bs, bt, sem):
    w = _worker_id()

    def do_chunk(ci):
        base = ci * _C
        pltpu.sync_copy(row_hbm.at[pl.ds(base, _C)], idx_r)
        pltpu.sync_copy(col_hbm.at[pl.ds(base, _C)], idx_c)
        c1 = pltpu.async_copy(h_hbm.at[idx_r], bs, sem)
        c2 = pltpu.async_copy(h_hbm.at[idx_c], bt, sem)
        c1.wait()
        c2.wait()
        pltpu.sync_copy(bs, src_o.at[pl.ds(base, _C)])
        pltpu.sync_copy(bt, tgt_o.at[pl.ds(base, _C)])

    def body(k, carry):
        do_chunk(w + k * _NW)
        return carry

    lax.fori_loop(0, _BASE_CH, body, 0)

    @pl.when(w < _EXTRA)
    def _():
        do_chunk(_BASE_CH * _NW + w)


def _gather2(h, row, col):
    f = functools.partial(
        pl.kernel, mesh=_mesh(),
        out_type=[
            jax.ShapeDtypeStruct((_E, _H), jnp.float32),
            jax.ShapeDtypeStruct((_E, _H), jnp.float32),
        ],
        scratch_types=[
            pltpu.VMEM((_C,), jnp.int32),
            pltpu.VMEM((_C,), jnp.int32),
            pltpu.VMEM((_C, _H), jnp.float32),
            pltpu.VMEM((_C, _H), jnp.float32),
            pltpu.SemaphoreType.DMA,
        ],
    )(_g2_body)
    return f(h, row, col)


def _make_scatter_body(width):
    def body_fn(ef_hbm, row_hbm, z_hbm, out_hbm, idx_v, buf, acc, sem):
        del sem
        c = lax.axis_index("c")
        s = lax.axis_index("s")
        w = s * _NC + c
        pltpu.sync_copy(z_hbm.at[pl.ds(s * _RPT, _RPT)],
                        acc.at[pl.ds(s * _RPT, _RPT)])
        plsc.subcore_barrier()

        def do_chunk(ci):
            base = ci * _C
            pltpu.sync_copy(row_hbm.at[pl.ds(base, _C)], idx_v)
            pltpu.sync_copy(ef_hbm.at[pl.ds(base, _C)], buf)
            pltpu.sync_copy(buf, acc.at[idx_v], add=True)

        def body(k, carry):
            do_chunk(w + k * _NW)
            return carry

        lax.fori_loop(0, _BASE_CH, body, 0)

        @pl.when(w < _EXTRA)
        def _():
            do_chunk(_BASE_CH * _NW + w)

        plsc.subcore_barrier()
        pltpu.sync_copy(acc.at[pl.ds(s * _RPT, _RPT)],
                        out_hbm.at[c, pl.ds(s * _RPT, _RPT)])
    return body_fn


def _scatter_add(ef, row, zeros, width):
    f = functools.partial(
        pl.kernel, mesh=_mesh(),
        out_type=jax.ShapeDtypeStruct((_NC, _N, width), jnp.float32),
        scratch_types=[
            pltpu.VMEM((_C,), jnp.int32),
            pltpu.VMEM((_C, width), jnp.float32),
            pltpu.VMEM_SHARED((_N, width), jnp.float32),
            pltpu.SemaphoreType.DMA,
        ],
    )(_make_scatter_body(width))
    return f(ef, row, zeros)


_BE = 1280
_BN = 1000


def _edge_body(src_ref, tgt_ref, cr_ref, cc_ref, ea_ref,
               A_ref, B_ref, b1_ref, ar_ref, ae_ref, g1_ref, bg1_ref,
               W2_ref, b2_ref, aw_ref, ab_ref, out_ref):
    d = cr_ref[...] - cc_ref[...]
    radial = jnp.sum(d * d, axis=1, keepdims=True)
    x = jnp.dot(src_ref[...], A_ref[...], preferred_element_type=jnp.float32)
    x = x + jnp.dot(tgt_ref[...], B_ref[...], preferred_element_type=jnp.float32)
    x = x + radial * ar_ref[...] + ea_ref[...] * ae_ref[...] + b1_ref[...]
    m = jnp.mean(x, axis=-1, keepdims=True)
    v = jnp.mean((x - m) ** 2, axis=-1, keepdims=True)
    x = (x - m) * lax.rsqrt(v + _EPS_LN) * g1_ref[...] + bg1_ref[...]
    x = x * jax.nn.sigmoid(x)
    y = jnp.dot(x, W2_ref[...], preferred_element_type=jnp.float32) + b2_ref[...]
    y = y * jax.nn.sigmoid(y)
    att = jax.nn.sigmoid(jnp.sum(y * aw_ref[...], axis=1, keepdims=True) + ab_ref[...])
    out_ref[...] = y * att


def _tc_edge(src, tgt, crow, ccol, ea, A, B, b1, ar, ae, g1, bg1, W2, b2, aw, ab):
    im = lambda i: (i, 0)
    full = lambda shape: pl.BlockSpec(shape, lambda i: (0, 0))
    return pl.pallas_call(
        _edge_body,
        grid=(_E // _BE,),
        in_specs=[
            pl.BlockSpec((_BE, _H), im), pl.BlockSpec((_BE, _H), im),
            pl.BlockSpec((_BE, 8), im), pl.BlockSpec((_BE, 8), im),
            pl.BlockSpec((_BE, 1), im),
            full((_H, _H)), full((_H, _H)), full((1, _H)), full((1, _H)),
            full((1, _H)), full((1, _H)), full((1, _H)),
            full((_H, _H)), full((1, _H)), full((1, _H)), full((1, 1)),
        ],
        out_specs=pl.BlockSpec((_BE, _H), im),
        out_shape=jax.ShapeDtypeStruct((_E, _H), jnp.float32),
    )(src, tgt, crow, ccol, ea, A, B, b1, ar, ae, g1, bg1, W2, b2, aw, ab)


def _node_body(h_ref, p0_ref, p1_ref,
               Wh_ref, Wa_ref, b1_ref, g_ref, bg_ref, W2_ref, b2_ref, out_ref):
    h = h_ref[...]
    agg = (p0_ref[...] + p1_ref[...]) * _NORM_INV
    x = jnp.dot(h, Wh_ref[...], preferred_element_type=jnp.float32)
    x = x + jnp.dot(agg, Wa_ref[...], preferred_element_type=jnp.float32) + b1_ref[...]
    m = jnp.mean(x, axis=-1, keepdims=True)
    v = jnp.mean((x - m) ** 2, axis=-1, keepdims=True)
    x = (x - m) * lax.rsqrt(v + _EPS_LN) * g_ref[...] + bg_ref[...]
    x = x * jax.nn.sigmoid(x)
    nu = jnp.dot(x, W2_ref[...], preferred_element_type=jnp.float32) + b2_ref[...]
    out_ref[...] = h + nu


def _tc_node(h, part, Wh, Wa, b1, g, bg, W2, b2):
    im = lambda i: (i, 0)
    full = lambda shape: pl.BlockSpec(shape, lambda i: (0, 0))
    return pl.pallas_call(
        _node_body,
        grid=(_N // _BN,),
        in_specs=[
            pl.BlockSpec((_BN, _H), im), pl.BlockSpec((_BN, _H), im),
            pl.BlockSpec((_BN, _H), im),
            full((_H, _H)), full((_H, _H)), full((1, _H)), full((1, _H)),
            full((1, _H)), full((_H, _H)), full((1, _H)),
        ],
        out_specs=pl.BlockSpec((_BN, _H), im),
        out_shape=jax.ShapeDtypeStruct((_N, _H), jnp.float32),
    )(h, part[0], part[1], Wh, Wa, b1, g, bg, W2, b2)


def _eq_body(src_ref, tgt_ref, cr_ref, cc_ref, ea_ref,
             A_ref, B_ref, b1_ref, ar_ref, ae_ref, g1_ref, bg1_ref,
             W2_ref, b2_ref, g2_ref, bg2_ref, w3_ref, out_ref):
    d = cr_ref[...] - cc_ref[...]
    radial = jnp.sum(d * d, axis=1, keepdims=True)
    x = jnp.dot(src_ref[...], A_ref[...], preferred_element_type=jnp.float32)
    x = x + jnp.dot(tgt_ref[...], B_ref[...], preferred_element_type=jnp.float32)
    x = x + radial * ar_ref[...] + ea_ref[...] * ae_ref[...] + b1_ref[...]
    m = jnp.mean(x, axis=-1, keepdims=True)
    v = jnp.mean((x - m) ** 2, axis=-1, keepdims=True)
    x = (x - m) * lax.rsqrt(v + _EPS_LN) * g1_ref[...] + bg1_ref[...]
    x = x * jax.nn.sigmoid(x)
    y = jnp.dot(x, W2_ref[...], preferred_element_type=jnp.float32) + b2_ref[...]
    m = jnp.mean(y, axis=-1, keepdims=True)
    v = jnp.mean((y - m) ** 2, axis=-1, keepdims=True)
    y = (y - m) * lax.rsqrt(v + _EPS_LN) * g2_ref[...] + bg2_ref[...]
    y = y * jax.nn.sigmoid(y)
    t = jnp.sum(y * w3_ref[...], axis=1, keepdims=True)
    cd = d / (jnp.sqrt(radial + _EPS_R) + 1.0)
    out_ref[...] = cd * t


def _tc_eq(src, tgt, crow, ccol, ea, A, B, b1, ar, ae, g1, bg1, W2, b2, g2, bg2, w3):
    im = lambda i: (i, 0)
    full = lambda shape: pl.BlockSpec(shape, lambda i: (0, 0))
    return pl.pallas_call(
        _eq_body,
        grid=(_E // _BE,),
        in_specs=[
            pl.BlockSpec((_BE, _H), im), pl.BlockSpec((_BE, _H), im),
            pl.BlockSpec((_BE, 8), im), pl.BlockSpec((_BE, 8), im),
            pl.BlockSpec((_BE, 1), im),
            full((_H, _H)), full((_H, _H)), full((1, _H)), full((1, _H)),
            full((1, _H)), full((1, _H)), full((1, _H)),
            full((_H, _H)), full((1, _H)), full((1, _H)), full((1, _H)),
            full((1, _H)),
        ],
        out_specs=pl.BlockSpec((_BE, 8), im),
        out_shape=jax.ShapeDtypeStruct((_E, 8), jnp.float32),
    )(src, tgt, crow, ccol, ea, A, B, b1, ar, ae, g1, bg1, W2, b2, g2, bg2, w3)


def _coord_body(cp_ref, p0_ref, p1_ref, out_ref):
    out_ref[...] = cp_ref[...] + (p0_ref[...] + p1_ref[...]) * _NORM_INV


def _tc_coord(cpad, part):
    im = lambda i: (i, 0)
    return pl.pallas_call(
        _coord_body,
        grid=(_N // _BN,),
        in_specs=[pl.BlockSpec((_BN, 8), im), pl.BlockSpec((_BN, 8), im),
                  pl.BlockSpec((_BN, 8), im)],
        out_specs=pl.BlockSpec((_BN, 8), im),
        out_shape=jax.ShapeDtypeStruct((_N, 8), jnp.float32),
    )(cpad, part[0], part[1])


def _edge_params(p, w1_key='e_w1', b1_key='e_b1', g_key='e_ln_g', bg_key='e_ln_b',
                 w2_key='e_w2', b2_key='e_b2'):
    W1 = p[w1_key]
    return dict(
        A=W1[:_H], B=W1[_H:2 * _H],
        ar=W1[2 * _H:2 * _H + 1], ae=W1[2 * _H + 1:2 * _H + 2],
        b1=p[b1_key].reshape(1, _H), g1=p[g_key].reshape(1, _H),
        bg1=p[bg_key].reshape(1, _H),
        W2=p[w2_key], b2=p[b2_key].reshape(1, _H),
    )


def kernel(h, coord, edge_attr, params, edge_index):
    row = edge_index[0]
    col = edge_index[1]
    cpad = jnp.pad(coord, ((0, 0), (0, 5)))
    zeros_h = jnp.zeros((_N, _H), jnp.float32)
    zeros_c = jnp.zeros((_N, 8), jnp.float32)

    src, tgt, crow, ccol = _gather4(h, cpad, row, col)

    for i in range(2):
        p = params['gcl%d' % i]
        ep = _edge_params(p)
        ef = _tc_edge(src, tgt, crow, ccol, edge_attr,
                      ep['A'], ep['B'], ep['b1'], ep['ar'], ep['ae'],
                      ep['g1'], ep['bg1'], ep['W2'], ep['b2'],
                      p['att_w'].reshape(1, _H), p['att_b'].reshape(1, 1))
        part = _scatter_add(ef, row, zeros_h, _H)
        h = _tc_node(h, part,
                     p['n_w1'][:_H], p['n_w1'][_H:], p['n_b1'].reshape(1, _H),
                     p['n_ln_g'].reshape(1, _H), p['n_ln_b'].reshape(1, _H),
                     p['n_w2'], p['n_b2'].reshape(1, _H))
        if i == 0:
            src, tgt = _gather2(h, row, col)

    src, tgt = _gather2(h, row, col)
    eq = params['eq']
    eqp = _edge_params(eq, w1_key='w1', b1_key='b1', g_key='ln1_g', bg_key='ln1_b',
                       w2_key='w2', b2_key='b2')
    trans = _tc_eq(src, tgt, crow, ccol, edge_attr,
                   eqp['A'], eqp['B'], eqp['b1'], eqp['ar'], eqp['ae'],
                   eqp['g1'], eqp['bg1'], eqp['W2'], eqp['b2'],
                   eq['ln2_g'].reshape(1, _H), eq['ln2_b'].reshape(1, _H),
                   eq['w3'].reshape(1, _H))
    partc = _scatter_add(trans, row, zeros_c, 8)
    cnew = _tc_coord(cpad, partc)
    return h, cnew[:, :3]


# speedup vs baseline: 3.2152x; 3.2152x over previous
import functools

import jax
import jax.numpy as jnp
from jax import lax
from jax.experimental import pallas as pl
from jax.experimental.pallas import tpu as pltpu
from jax.experimental.pallas import tpu_sc as plsc

_N = 10000
_E = 320000
_H = 128
_NORM_INV = 0.01
_EPS_LN = 1e-5
_EPS_R = 1e-8

_NC = 2
_NS = 16
_NW = _NC * _NS
_L = 16
_C = 128
_NCH = _E // _C
_BASE_CH = _NCH // _NW
_EXTRA = _NCH - _BASE_CH * _NW
_RPT = 624
_CW = 8


def _mesh():
    return plsc.VectorSubcoreMesh(core_axis_name="c", subcore_axis_name="s")


def _worker_id():
    return lax.axis_index("s") * _NC + lax.axis_index("c")


def _foreach_chunk(do_chunk):
    w = _worker_id()

    def body(k, carry):
        do_chunk(w + k * _NW)
        return carry

    lax.fori_loop(0, _BASE_CH, body, 0)

    @pl.when(w < _EXTRA)
    def _():
        do_chunk(_BASE_CH * _NW + w)


def _geom_body(ct_hbm, row_hbm, col_hbm, geom_o, idx_r, idx_c, ct_v, bg, sem):
    del sem
    pltpu.sync_copy(ct_hbm, ct_v)
    lanes = jnp.arange(_L, dtype=jnp.int32)

    def do_chunk(ci):
        base = ci * _C
        pltpu.sync_copy(row_hbm.at[pl.ds(base, _C)], idx_r)
        pltpu.sync_copy(col_hbm.at[pl.ds(base, _C)], idx_c)
        for j in range(_C // _L):
            r16 = idx_r[pl.ds(j * _L, _L)] * _CW
            c16 = idx_c[pl.ds(j * _L, _L)] * _CW
            a0 = plsc.load_gather(ct_v, [r16])
            a1 = plsc.load_gather(ct_v, [r16 + 1])
            a2 = plsc.load_gather(ct_v, [r16 + 2])
            b0 = plsc.load_gather(ct_v, [c16])
            b1 = plsc.load_gather(ct_v, [c16 + 1])
            b2 = plsc.load_gather(ct_v, [c16 + 2])
            d0 = a0 - b0
            d1 = a1 - b1
            d2 = a2 - b2
            rad = d0 * d0 + d1 * d1 + d2 * d2
            eb = (j * _L + lanes) * _CW
            plsc.store_scatter(bg, [eb], d0)
            plsc.store_scatter(bg, [eb + 1], d1)
            plsc.store_scatter(bg, [eb + 2], d2)
            plsc.store_scatter(bg, [eb + 3], rad)
        pltpu.sync_copy(bg, geom_o.at[pl.ds(base * _CW, _C * _CW)])

    _foreach_chunk(do_chunk)


def _sc_geom(ctab, row, col):
    f = functools.partial(
        pl.kernel, mesh=_mesh(),
        out_type=jax.ShapeDtypeStruct((_E * _CW,), jnp.float32),
        compiler_params=pltpu.CompilerParams(needs_layout_passes=False),
        scratch_types=[
            pltpu.VMEM((_C,), jnp.int32),
            pltpu.VMEM((_C,), jnp.int32),
            pltpu.VMEM((_N * _CW,), jnp.float32),
            pltpu.VMEM((_C * _CW,), jnp.float32),
            pltpu.SemaphoreType.DMA,
        ],
    )(_geom_body)
    return f(ctab, row, col)


def _g2_body(h_hbm, row_hbm, col_hbm, src_o, tgt_o,
             idx_r, idx_c, bs, bt, sem):
    def do_chunk(ci):
        base = ci * _C
        pltpu.sync_copy(row_hbm.at[pl.ds(base, _C)], idx_r)
        pltpu.sync_copy(col_hbm.at[pl.ds(base, _C)], idx_c)
        c1 = pltpu.async_copy(h_hbm.at[idx_r], bs, sem)
        c2 = pltpu.async_copy(h_hbm.at[idx_c], bt, sem)
        c1.wait()
        c2.wait()
        pltpu.sync_copy(bs, src_o.at[pl.ds(base, _C)])
        pltpu.sync_copy(bt, tgt_o.at[pl.ds(base, _C)])

    _foreach_chunk(do_chunk)


def _gather2(h, row, col):
    f = functools.partial(
        pl.kernel, mesh=_mesh(),
        out_type=[
            jax.ShapeDtypeStruct((_E, _H), jnp.float32),
            jax.ShapeDtypeStruct((_E, _H), jnp.float32),
        ],
        scratch_types=[
            pltpu.VMEM((_C,), jnp.int32),
            pltpu.VMEM((_C,), jnp.int32),
            pltpu.VMEM((_C, _H), jnp.float32),
            pltpu.VMEM((_C, _H), jnp.float32),
            pltpu.SemaphoreType.DMA,
        ],
    )(_g2_body)
    return f(h, row, col)


def _scatter_body(ef_hbm, row_hbm, z_hbm, out_hbm, idx_v, buf, acc, sem):
    del sem
    c = lax.axis_index("c")
    s = lax.axis_index("s")
    start = s * _RPT

    @pl.when(s < _NS - 1)
    def _():
        pltpu.sync_copy(z_hbm.at[pl.ds(start, _RPT)],
                        acc.at[pl.ds(start, _RPT)])

    @pl.when(s == _NS - 1)
    def _():
        pltpu.sync_copy(z_hbm.at[pl.ds((_NS - 1) * _RPT, _N - (_NS - 1) * _RPT)],
                        acc.at[pl.ds((_NS - 1) * _RPT, _N - (_NS - 1) * _RPT)])

    plsc.subcore_barrier()

    def do_chunk(ci):
        base = ci * _C
        pltpu.sync_copy(row_hbm.at[pl.ds(base, _C)], idx_v)
        pltpu.sync_copy(ef_hbm.at[pl.ds(base, _C)], buf)
        pltpu.sync_copy(buf, acc.at[idx_v], add=True)

    _foreach_chunk(do_chunk)

    plsc.subcore_barrier()

    @pl.when(s < _NS - 1)
    def _():
        pltpu.sync_copy(acc.at[pl.ds(start, _RPT)],
                        out_hbm.at[c, pl.ds(start, _RPT)])

    @pl.when(s == _NS - 1)
    def _():
        pltpu.sync_copy(acc.at[pl.ds((_NS - 1) * _RPT, _N - (_NS - 1) * _RPT)],
                        out_hbm.at[c, pl.ds((_NS - 1) * _RPT, _N - (_NS - 1) * _RPT)])


def _scatter_add(ef, row, zeros):
    f = functools.partial(
        pl.kernel, mesh=_mesh(),
        out_type=jax.ShapeDtypeStruct((_NC, _N, _H), jnp.float32),
        scratch_types=[
            pltpu.VMEM((_C,), jnp.int32),
            pltpu.VMEM((_C, _H), jnp.float32),
            pltpu.VMEM_SHARED((_N, _H), jnp.float32),
            pltpu.SemaphoreType.DMA,
        ],
    )(_scatter_body)
    return f(ef, row, zeros)


_BE = 1280
_BN = 1000


def _edge_body(src_ref, tgt_ref, geom_ref, ea_ref,
               A_ref, B_ref, b1_ref, ar_ref, ae_ref, g1_ref, bg1_ref,
               W2_ref, b2_ref, aw_ref, ab_ref, out_ref):
    radial = geom_ref[...][:, 3:4]
    x = jnp.dot(src_ref[...], A_ref[...], preferred_element_type=jnp.float32)
    x = x + jnp.dot(tgt_ref[...], B_ref[...], preferred_element_type=jnp.float32)
    x = x + radial * ar_ref[...] + ea_ref[...] * ae_ref[...] + b1_ref[...]
    m = jnp.mean(x, axis=-1, keepdims=True)
    v = jnp.mean((x - m) ** 2, axis=-1, keepdims=True)
    x = (x - m) * lax.rsqrt(v + _EPS_LN) * g1_ref[...] + bg1_ref[...]
    x = x * jax.nn.sigmoid(x)
    y = jnp.dot(x, W2_ref[...], preferred_element_type=jnp.float32) + b2_ref[...]
    y = y * jax.nn.sigmoid(y)
    att = jax.nn.sigmoid(jnp.sum(y * aw_ref[...], axis=1, keepdims=True) + ab_ref[...])
    out_ref[...] = y * att


def _tc_edge(src, tgt, geom, ea, A, B, b1, ar, ae, g1, bg1, W2, b2, aw, ab):
    im = lambda i: (i, 0)
    full = lambda shape: pl.BlockSpec(shape, lambda i: (0, 0))
    return pl.pallas_call(
        _edge_body,
        grid=(_E // _BE,),
        in_specs=[
            pl.BlockSpec((_BE, _H), im), pl.BlockSpec((_BE, _H), im),
            pl.BlockSpec((_BE, _CW), im), pl.BlockSpec((_BE, 1), im),
            full((_H, _H)), full((_H, _H)), full((1, _H)), full((1, _H)),
            full((1, _H)), full((1, _H)), full((1, _H)),
            full((_H, _H)), full((1, _H)), full((1, _H)), full((1, 1)),
        ],
        out_specs=pl.BlockSpec((_BE, _H), im),
        out_shape=jax.ShapeDtypeStruct((_E, _H), jnp.float32),
    )(src, tgt, geom, ea, A, B, b1, ar, ae, g1, bg1, W2, b2, aw, ab)


def _node_body(h_ref, p0_ref, p1_ref,
               Wh_ref, Wa_ref, b1_ref, g_ref, bg_ref, W2_ref, b2_ref, out_ref):
    h = h_ref[...]
    agg = (p0_ref[...] + p1_ref[...]) * _NORM_INV
    x = jnp.dot(h, Wh_ref[...], preferred_element_type=jnp.float32)
    x = x + jnp.dot(agg, Wa_ref[...], preferred_element_type=jnp.float32) + b1_ref[...]
    m = jnp.mean(x, axis=-1, keepdims=True)
    v = jnp.mean((x - m) ** 2, axis=-1, keepdims=True)
    x = (x - m) * lax.rsqrt(v + _EPS_LN) * g_ref[...] + bg_ref[...]
    x = x * jax.nn.sigmoid(x)
    nu = jnp.dot(x, W2_ref[...], preferred_element_type=jnp.float32) + b2_ref[...]
    out_ref[...] = h + nu


def _tc_node(h, part, Wh, Wa, b1, g, bg, W2, b2):
    im = lambda i: (i, 0)
    full = lambda shape: pl.BlockSpec(shape, lambda i: (0, 0))
    return pl.pallas_call(
        _node_body,
        grid=(_N // _BN,),
        in_specs=[
            pl.BlockSpec((_BN, _H), im), pl.BlockSpec((_BN, _H), im),
            pl.BlockSpec((_BN, _H), im),
            full((_H, _H)), full((_H, _H)), full((1, _H)), full((1, _H)),
            full((1, _H)), full((_H, _H)), full((1, _H)),
        ],
        out_specs=pl.BlockSpec((_BN, _H), im),
        out_shape=jax.ShapeDtypeStruct((_N, _H), jnp.float32),
    )(h, part[0], part[1], Wh, Wa, b1, g, bg, W2, b2)


def _eq_body(src_ref, tgt_ref, geom_ref, ea_ref,
             A_ref, B_ref, b1_ref, ar_ref, ae_ref, g1_ref, bg1_ref,
             W2_ref, b2_ref, g2_ref, bg2_ref, w3_ref, out_ref):
    geom = geom_ref[...]
    radial = geom[:, 3:4]
    x = jnp.dot(src_ref[...], A_ref[...], preferred_element_type=jnp.float32)
    x = x + jnp.dot(tgt_ref[...], B_ref[...], preferred_element_type=jnp.float32)
    x = x + radial * ar_ref[...] + ea_ref[...] * ae_ref[...] + b1_ref[...]
    m = jnp.mean(x, axis=-1, keepdims=True)
    v = jnp.mean((x - m) ** 2, axis=-1, keepdims=True)
    x = (x - m) * lax.rsqrt(v + _EPS_LN) * g1_ref[...] + bg1_ref[...]
    x = x * jax.nn.sigmoid(x)
    y = jnp.dot(x, W2_ref[...], preferred_element_type=jnp.float32) + b2_ref[...]
    m = jnp.mean(y, axis=-1, keepdims=True)
    v = jnp.mean((y - m) ** 2, axis=-1, keepdims=True)
    y = (y - m) * lax.rsqrt(v + _EPS_LN) * g2_ref[...] + bg2_ref[...]
    y = y * jax.nn.sigmoid(y)
    t = jnp.sum(y * w3_ref[...], axis=1, keepdims=True)
    mask = (lax.broadcasted_iota(jnp.int32, (1, _CW), 1) < 3).astype(jnp.float32)
    cd = geom * mask * (t / (jnp.sqrt(radial + _EPS_R) + 1.0))
    out_ref[...] = jnp.concatenate(
        [cd, jnp.zeros((cd.shape[0], _H - _CW), jnp.float32)], axis=1)


def _tc_eq(src, tgt, geom, ea, A, B, b1, ar, ae, g1, bg1, W2, b2, g2, bg2, w3):
    im = lambda i: (i, 0)
    full = lambda shape: pl.BlockSpec(shape, lambda i: (0, 0))
    return pl.pallas_call(
        _eq_body,
        grid=(_E // _BE,),
        in_specs=[
            pl.BlockSpec((_BE, _H), im), pl.BlockSpec((_BE, _H), im),
            pl.BlockSpec((_BE, _CW), im), pl.BlockSpec((_BE, 1), im),
            full((_H, _H)), full((_H, _H)), full((1, _H)), full((1, _H)),
            full((1, _H)), full((1, _H)), full((1, _H)),
            full((_H, _H)), full((1, _H)), full((1, _H)), full((1, _H)),
            full((1, _H)),
        ],
        out_specs=pl.BlockSpec((_BE, _H), im),
        out_shape=jax.ShapeDtypeStruct((_E, _H), jnp.float32),
    )(src, tgt, geom, ea, A, B, b1, ar, ae, g1, bg1, W2, b2, g2, bg2, w3)


def _coord_body(cp_ref, p0_ref, p1_ref, out_ref):
    out_ref[...] = cp_ref[...] + (p0_ref[...] + p1_ref[...]) * _NORM_INV


def _tc_coord(cpad, part):
    im = lambda i: (i, 0)
    return pl.pallas_call(
        _coord_body,
        grid=(_N // _BN,),
        in_specs=[pl.BlockSpec((_BN, _H), im), pl.BlockSpec((_BN, _H), im),
                  pl.BlockSpec((_BN, _H), im)],
        out_specs=pl.BlockSpec((_BN, _H), im),
        out_shape=jax.ShapeDtypeStruct((_N, _H), jnp.float32),
    )(cpad, part[0], part[1])


def _edge_params(p, w1_key='e_w1', b1_key='e_b1', g_key='e_ln_g', bg_key='e_ln_b',
                 w2_key='e_w2', b2_key='e_b2'):
    W1 = p[w1_key]
    return dict(
        A=W1[:_H], B=W1[_H:2 * _H],
        ar=W1[2 * _H:2 * _H + 1], ae=W1[2 * _H + 1:2 * _H + 2],
        b1=p[b1_key].reshape(1, _H), g1=p[g_key].reshape(1, _H),
        bg1=p[bg_key].reshape(1, _H),
        W2=p[w2_key], b2=p[b2_key].reshape(1, _H),
    )


def kernel(h, coord, edge_attr, params, edge_index):
    row = edge_index[0]
    col = edge_index[1]
    ctab = jnp.pad(coord, ((0, 0), (0, _CW - 3))).reshape(-1)
    cpad128 = jnp.pad(coord, ((0, 0), (0, _H - 3)))
    zeros_h = jnp.zeros((_N, _H), jnp.float32)

    geom = _sc_geom(ctab, row, col).reshape(_E, _CW)

    for i in range(2):
        p = params['gcl%d' % i]
        ep = _edge_params(p)
        src, tgt = _gather2(h, row, col)
        ef = _tc_edge(src, tgt, geom, edge_attr,
                      ep['A'], ep['B'], ep['b1'], ep['ar'], ep['ae'],
                      ep['g1'], ep['bg1'], ep['W2'], ep['b2'],
                      p['att_w'].reshape(1, _H), p['att_b'].reshape(1, 1))
        part = _scatter_add(ef, row, zeros_h)
        h = _tc_node(h, part,
                     p['n_w1'][:_H], p['n_w1'][_H:], p['n_b1'].reshape(1, _H),
                     p['n_ln_g'].reshape(1, _H), p['n_ln_b'].reshape(1, _H),
                     p['n_w2'], p['n_b2'].reshape(1, _H))

    src, tgt = _gather2(h, row, col)
    eq = params['eq']
    eqp = _edge_params(eq, w1_key='w1', b1_key='b1', g_key='ln1_g', bg_key='ln1_b',
                       w2_key='w2', b2_key='b2')
    trans = _tc_eq(src, tgt, geom, edge_attr,
                   eqp['A'], eqp['B'], eqp['b1'], eqp['ar'], eqp['ae'],
                   eqp['g1'], eqp['bg1'], eqp['W2'], eqp['b2'],
                   eq['ln2_g'].reshape(1, _H), eq['ln2_b'].reshape(1, _H),
                   eq['w3'].reshape(1, _H))
    partc = _scatter_add(trans, row, zeros_h)
    cnew = _tc_coord(cpad128, partc)
    return h, cnew[:, :3]


# speedup vs baseline: 3.9205x; 1.2194x over previous
import functools

import jax
import jax.numpy as jnp
from jax import lax
from jax.experimental import pallas as pl
from jax.experimental.pallas import tpu as pltpu
from jax.experimental.pallas import tpu_sc as plsc

_N = 10000
_E = 320000
_H = 128
_NORM_INV = 0.01
_EPS_LN = 1e-5
_EPS_R = 1e-8

_NC = 2
_NS = 16
_NW = _NC * _NS
_L = 16
_C = 128
_NCH = _E // _C
_BASE_CH = _NCH // _NW
_EXTRA = _NCH - _BASE_CH * _NW
_RPT = 624
_CW = 8


def _mesh():
    return plsc.VectorSubcoreMesh(core_axis_name="c", subcore_axis_name="s")


def _worker_id():
    return lax.axis_index("s") * _NC + lax.axis_index("c")


def _foreach_chunk(do_chunk):
    w = _worker_id()

    def body(k, carry):
        do_chunk(w + k * _NW)
        return carry

    lax.fori_loop(0, _BASE_CH, body, 0)

    @pl.when(w < _EXTRA)
    def _():
        do_chunk(_BASE_CH * _NW + w)


def _geom_body(ct_hbm, row_hbm, col_hbm, geom_o, idx_r, idx_c, ct_v, bg, sem):
    del sem
    pltpu.sync_copy(ct_hbm, ct_v)
    lanes = jnp.arange(_L, dtype=jnp.int32)

    def do_chunk(ci):
        base = ci * _C
        pltpu.sync_copy(row_hbm.at[pl.ds(base, _C)], idx_r)
        pltpu.sync_copy(col_hbm.at[pl.ds(base, _C)], idx_c)
        for j in range(_C // _L):
            r16 = idx_r[pl.ds(j * _L, _L)] * _CW
            c16 = idx_c[pl.ds(j * _L, _L)] * _CW
            a0 = plsc.load_gather(ct_v, [r16])
            a1 = plsc.load_gather(ct_v, [r16 + 1])
            a2 = plsc.load_gather(ct_v, [r16 + 2])
            b0 = plsc.load_gather(ct_v, [c16])
            b1 = plsc.load_gather(ct_v, [c16 + 1])
            b2 = plsc.load_gather(ct_v, [c16 + 2])
            d0 = a0 - b0
            d1 = a1 - b1
            d2 = a2 - b2
            rad = d0 * d0 + d1 * d1 + d2 * d2
            eb = (j * _L + lanes) * _CW
            plsc.store_scatter(bg, [eb], d0)
            plsc.store_scatter(bg, [eb + 1], d1)
            plsc.store_scatter(bg, [eb + 2], d2)
            plsc.store_scatter(bg, [eb + 3], rad)
        pltpu.sync_copy(bg, geom_o.at[pl.ds(base * _CW, _C * _CW)])

    _foreach_chunk(do_chunk)


def _sc_geom(ctab, row, col):
    f = functools.partial(
        pl.kernel, mesh=_mesh(),
        out_type=jax.ShapeDtypeStruct((_E * _CW,), jnp.float32),
        compiler_params=pltpu.CompilerParams(needs_layout_passes=False),
        scratch_types=[
            pltpu.VMEM((_C,), jnp.int32),
            pltpu.VMEM((_C,), jnp.int32),
            pltpu.VMEM((_N * _CW,), jnp.float32),
            pltpu.VMEM((_C * _CW,), jnp.float32),
            pltpu.SemaphoreType.DMA,
        ],
    )(_geom_body)
    return f(ctab, row, col)


_NB = 3
_GRP = _BASE_CH // _NB


def _g2_body(h_hbm, row_hbm, col_hbm, src_o, tgt_o,
             idx_r, idx_c, bs, bt, gsem, ws0, ws1, ws2):
    w = _worker_id()
    wsems = (ws0, ws1, ws2)

    def drain(j):
        pltpu.make_async_copy(bs.at[j], src_o.at[pl.ds(0, _C)], wsems[j]).wait()
        pltpu.make_async_copy(bt.at[j], tgt_o.at[pl.ds(0, _C)], wsems[j]).wait()

    def body(m, carry):
        handles = []
        for j in range(_NB):
            ci = w + (_NB * m + j) * _NW
            base = ci * _C

            @pl.when(m > 0)
            def _():
                drain(j)

            pltpu.sync_copy(row_hbm.at[pl.ds(base, _C)], idx_r.at[j])
            pltpu.sync_copy(col_hbm.at[pl.ds(base, _C)], idx_c.at[j])
            g1 = pltpu.async_copy(h_hbm.at[idx_r.at[j]], bs.at[j], gsem)
            g2 = pltpu.async_copy(h_hbm.at[idx_c.at[j]], bt.at[j], gsem)
            handles.append((g1, g2, base))
        for j in range(_NB):
            g1, g2, base = handles[j]
            g1.wait()
            g2.wait()
            pltpu.async_copy(bs.at[j], src_o.at[pl.ds(base, _C)], wsems[j])
            pltpu.async_copy(bt.at[j], tgt_o.at[pl.ds(base, _C)], wsems[j])
        return carry

    lax.fori_loop(0, _GRP, body, 0)
    for j in range(_NB):
        drain(j)

    @pl.when(w < _EXTRA)
    def _():
        base = (_BASE_CH * _NW + w) * _C
        pltpu.sync_copy(row_hbm.at[pl.ds(base, _C)], idx_r.at[0])
        pltpu.sync_copy(col_hbm.at[pl.ds(base, _C)], idx_c.at[0])
        g1 = pltpu.async_copy(h_hbm.at[idx_r.at[0]], bs.at[0], gsem)
        g2 = pltpu.async_copy(h_hbm.at[idx_c.at[0]], bt.at[0], gsem)
        g1.wait()
        g2.wait()
        pltpu.sync_copy(bs.at[0], src_o.at[pl.ds(base, _C)])
        pltpu.sync_copy(bt.at[0], tgt_o.at[pl.ds(base, _C)])


def _gather2(h, row, col):
    f = functools.partial(
        pl.kernel, mesh=_mesh(),
        out_type=[
            jax.ShapeDtypeStruct((_E, _H), jnp.float32),
            jax.ShapeDtypeStruct((_E, _H), jnp.float32),
        ],
        scratch_types=[
            pltpu.VMEM((_NB, _C), jnp.int32),
            pltpu.VMEM((_NB, _C), jnp.int32),
            pltpu.VMEM((_NB, _C, _H), jnp.float32),
            pltpu.VMEM((_NB, _C, _H), jnp.float32),
            pltpu.SemaphoreType.DMA,
            pltpu.SemaphoreType.DMA,
            pltpu.SemaphoreType.DMA,
            pltpu.SemaphoreType.DMA,
        ],
    )(_g2_body)
    return f(h, row, col)


def _scatter_body(ef_hbm, row_hbm, z_hbm, out_hbm, idx_v, buf, acc,
                  lsem, ss0, ss1, ss2):
    c = lax.axis_index("c")
    s = lax.axis_index("s")
    start = s * _RPT

    @pl.when(s < _NS - 1)
    def _():
        pltpu.sync_copy(z_hbm.at[pl.ds(start, _RPT)],
                        acc.at[pl.ds(start, _RPT)])

    @pl.when(s == _NS - 1)
    def _():
        pltpu.sync_copy(z_hbm.at[pl.ds((_NS - 1) * _RPT, _N - (_NS - 1) * _RPT)],
                        acc.at[pl.ds((_NS - 1) * _RPT, _N - (_NS - 1) * _RPT)])

    plsc.subcore_barrier()

    w = s * _NC + c
    ssems = (ss0, ss1, ss2)

    def drain(j):
        pltpu.make_async_copy(buf.at[j], acc.at[pl.ds(0, _C)], ssems[j]).wait()

    def body(m, carry):
        handles = []
        for j in range(_NB):
            ci = w + (_NB * m + j) * _NW
            base = ci * _C

            @pl.when(m > 0)
            def _():
                drain(j)

            pltpu.sync_copy(row_hbm.at[pl.ds(base, _C)], idx_v.at[j])
            handles.append(
                pltpu.async_copy(ef_hbm.at[pl.ds(base, _C)], buf.at[j], lsem))
        for j in range(_NB):
            handles[j].wait()
            pltpu.async_copy(buf.at[j], acc.at[idx_v.at[j]], ssems[j], add=True)
        return carry

    lax.fori_loop(0, _GRP, body, 0)
    for j in range(_NB):
        drain(j)

    @pl.when(w < _EXTRA)
    def _():
        base = (_BASE_CH * _NW + w) * _C
        pltpu.sync_copy(row_hbm.at[pl.ds(base, _C)], idx_v.at[0])
        pltpu.sync_copy(ef_hbm.at[pl.ds(base, _C)], buf.at[0])
        pltpu.sync_copy(buf.at[0], acc.at[idx_v.at[0]], add=True)

    plsc.subcore_barrier()

    @pl.when(s < _NS - 1)
    def _():
        pltpu.sync_copy(acc.at[pl.ds(start, _RPT)],
                        out_hbm.at[c, pl.ds(start, _RPT)])

    @pl.when(s == _NS - 1)
    def _():
        pltpu.sync_copy(acc.at[pl.ds((_NS - 1) * _RPT, _N - (_NS - 1) * _RPT)],
                        out_hbm.at[c, pl.ds((_NS - 1) * _RPT, _N - (_NS - 1) * _RPT)])


def _scatter_add(ef, row, zeros):
    f = functools.partial(
        pl.kernel, mesh=_mesh(),
        out_type=jax.ShapeDtypeStruct((_NC, _N, _H), jnp.float32),
        scratch_types=[
            pltpu.VMEM((_NB, _C), jnp.int32),
            pltpu.VMEM((_NB, _C, _H), jnp.float32),
            pltpu.VMEM_SHARED((_N, _H), jnp.float32),
            pltpu.SemaphoreType.DMA,
            pltpu.SemaphoreType.DMA,
            pltpu.SemaphoreType.DMA,
            pltpu.SemaphoreType.DMA,
        ],
    )(_scatter_body)
    return f(ef, row, zeros)


_BE = 1280
_BN = 1000


def _edge_body(src_ref, tgt_ref, geom_ref, ea_ref,
               A_ref, B_ref, b1_ref, ar_ref, ae_ref, g1_ref, bg1_ref,
               W2_ref, b2_ref, aw_ref, ab_ref, out_ref):
    radial = geom_ref[...][:, 3:4]
    x = jnp.dot(src_ref[...], A_ref[...], preferred_element_type=jnp.float32)
    x = x + jnp.dot(tgt_ref[...], B_ref[...], preferred_element_type=jnp.float32)
    x = x + radial * ar_ref[...] + ea_ref[...] * ae_ref[...] + b1_ref[...]
    m = jnp.mean(x, axis=-1, keepdims=True)
    v = jnp.mean((x - m) ** 2, axis=-1, keepdims=True)
    x = (x - m) * lax.rsqrt(v + _EPS_LN) * g1_ref[...] + bg1_ref[...]
    x = x * jax.nn.sigmoid(x)
    y = jnp.dot(x, W2_ref[...], preferred_element_type=jnp.float32) + b2_ref[...]
    y = y * jax.nn.sigmoid(y)
    att = jax.nn.sigmoid(jnp.sum(y * aw_ref[...], axis=1, keepdims=True) + ab_ref[...])
    out_ref[...] = y * att


def _tc_edge(src, tgt, geom, ea, A, B, b1, ar, ae, g1, bg1, W2, b2, aw, ab):
    im = lambda i: (i, 0)
    full = lambda shape: pl.BlockSpec(shape, lambda i: (0, 0))
    return pl.pallas_call(
        _edge_body,
        grid=(_E // _BE,),
        in_specs=[
            pl.BlockSpec((_BE, _H), im), pl.BlockSpec((_BE, _H), im),
            pl.BlockSpec((_BE, _CW), im), pl.BlockSpec((_BE, 1), im),
            full((_H, _H)), full((_H, _H)), full((1, _H)), full((1, _H)),
            full((1, _H)), full((1, _H)), full((1, _H)),
            full((_H, _H)), full((1, _H)), full((1, _H)), full((1, 1)),
        ],
        out_specs=pl.BlockSpec((_BE, _H), im),
        out_shape=jax.ShapeDtypeStruct((_E, _H), jnp.float32),
    )(src, tgt, geom, ea, A, B, b1, ar, ae, g1, bg1, W2, b2, aw, ab)


def _node_body(h_ref, p0_ref, p1_ref,
               Wh_ref, Wa_ref, b1_ref, g_ref, bg_ref, W2_ref, b2_ref, out_ref):
    h = h_ref[...]
    agg = (p0_ref[...] + p1_ref[...]) * _NORM_INV
    x = jnp.dot(h, Wh_ref[...], preferred_element_type=jnp.float32)
    x = x + jnp.dot(agg, Wa_ref[...], preferred_element_type=jnp.float32) + b1_ref[...]
    m = jnp.mean(x, axis=-1, keepdims=True)
    v = jnp.mean((x - m) ** 2, axis=-1, keepdims=True)
    x = (x - m) * lax.rsqrt(v + _EPS_LN) * g_ref[...] + bg_ref[...]
    x = x * jax.nn.sigmoid(x)
    nu = jnp.dot(x, W2_ref[...], preferred_element_type=jnp.float32) + b2_ref[...]
    out_ref[...] = h + nu


def _tc_node(h, part, Wh, Wa, b1, g, bg, W2, b2):
    im = lambda i: (i, 0)
    full = lambda shape: pl.BlockSpec(shape, lambda i: (0, 0))
    return pl.pallas_call(
        _node_body,
        grid=(_N // _BN,),
        in_specs=[
            pl.BlockSpec((_BN, _H), im), pl.BlockSpec((_BN, _H), im),
            pl.BlockSpec((_BN, _H), im),
            full((_H, _H)), full((_H, _H)), full((1, _H)), full((1, _H)),
            full((1, _H)), full((_H, _H)), full((1, _H)),
        ],
        out_specs=pl.BlockSpec((_BN, _H), im),
        out_shape=jax.ShapeDtypeStruct((_N, _H), jnp.float32),
    )(h, part[0], part[1], Wh, Wa, b1, g, bg, W2, b2)


def _eq_body(src_ref, tgt_ref, geom_ref, ea_ref,
             A_ref, B_ref, b1_ref, ar_ref, ae_ref, g1_ref, bg1_ref,
             W2_ref, b2_ref, g2_ref, bg2_ref, w3_ref, out_ref):
    geom = geom_ref[...]
    radial = geom[:, 3:4]
    x = jnp.dot(src_ref[...], A_ref[...], preferred_element_type=jnp.float32)
    x = x + jnp.dot(tgt_ref[...], B_ref[...], preferred_element_type=jnp.float32)
    x = x + radial * ar_ref[...] + ea_ref[...] * ae_ref[...] + b1_ref[...]
    m = jnp.mean(x, axis=-1, keepdims=True)
    v = jnp.mean((x - m) ** 2, axis=-1, keepdims=True)
    x = (x - m) * lax.rsqrt(v + _EPS_LN) * g1_ref[...] + bg1_ref[...]
    x = x * jax.nn.sigmoid(x)
    y = jnp.dot(x, W2_ref[...], preferred_element_type=jnp.float32) + b2_ref[...]
    m = jnp.mean(y, axis=-1, keepdims=True)
    v = jnp.mean((y - m) ** 2, axis=-1, keepdims=True)
    y = (y - m) * lax.rsqrt(v + _EPS_LN) * g2_ref[...] + bg2_ref[...]
    y = y * jax.nn.sigmoid(y)
    t = jnp.sum(y * w3_ref[...], axis=1, keepdims=True)
    mask = (lax.broadcasted_iota(jnp.int32, (1, _CW), 1) < 3).astype(jnp.float32)
    cd = geom * mask * (t / (jnp.sqrt(radial + _EPS_R) + 1.0))
    out_ref[...] = jnp.concatenate(
        [cd, jnp.zeros((cd.shape[0], _H - _CW), jnp.float32)], axis=1)


def _tc_eq(src, tgt, geom, ea, A, B, b1, ar, ae, g1, bg1, W2, b2, g2, bg2, w3):
    im = lambda i: (i, 0)
    full = lambda shape: pl.BlockSpec(shape, lambda i: (0, 0))
    return pl.pallas_call(
        _eq_body,
        grid=(_E // _BE,),
        in_specs=[
            pl.BlockSpec((_BE, _H), im), pl.BlockSpec((_BE, _H), im),
            pl.BlockSpec((_BE, _CW), im), pl.BlockSpec((_BE, 1), im),
            full((_H, _H)), full((_H, _H)), full((1, _H)), full((1, _H)),
            full((1, _H)), full((1, _H)), full((1, _H)),
            full((_H, _H)), full((1, _H)), full((1, _H)), full((1, _H)),
            full((1, _H)),
        ],
        out_specs=pl.BlockSpec((_BE, _H), im),
        out_shape=jax.ShapeDtypeStruct((_E, _H), jnp.float32),
    )(src, tgt, geom, ea, A, B, b1, ar, ae, g1, bg1, W2, b2, g2, bg2, w3)


def _coord_body(cp_ref, p0_ref, p1_ref, out_ref):
    out_ref[...] = cp_ref[...] + (p0_ref[...] + p1_ref[...]) * _NORM_INV


def _tc_coord(cpad, part):
    im = lambda i: (i, 0)
    return pl.pallas_call(
        _coord_body,
        grid=(_N // _BN,),
        in_specs=[pl.BlockSpec((_BN, _H), im), pl.BlockSpec((_BN, _H), im),
                  pl.BlockSpec((_BN, _H), im)],
        out_specs=pl.BlockSpec((_BN, _H), im),
        out_shape=jax.ShapeDtypeStruct((_N, _H), jnp.float32),
    )(cpad, part[0], part[1])


def _edge_params(p, w1_key='e_w1', b1_key='e_b1', g_key='e_ln_g', bg_key='e_ln_b',
                 w2_key='e_w2', b2_key='e_b2'):
    W1 = p[w1_key]
    return dict(
        A=W1[:_H], B=W1[_H:2 * _H],
        ar=W1[2 * _H:2 * _H + 1], ae=W1[2 * _H + 1:2 * _H + 2],
        b1=p[b1_key].reshape(1, _H), g1=p[g_key].reshape(1, _H),
        bg1=p[bg_key].reshape(1, _H),
        W2=p[w2_key], b2=p[b2_key].reshape(1, _H),
    )


def kernel(h, coord, edge_attr, params, edge_index):
    row = edge_index[0]
    col = edge_index[1]
    ctab = jnp.pad(coord, ((0, 0), (0, _CW - 3))).reshape(-1)
    cpad128 = jnp.pad(coord, ((0, 0), (0, _H - 3)))
    zeros_h = jnp.zeros((_N, _H), jnp.float32)

    geom = _sc_geom(ctab, row, col).reshape(_E, _CW)

    for i in range(2):
        p = params['gcl%d' % i]
        ep = _edge_params(p)
        src, tgt = _gather2(h, row, col)
        ef = _tc_edge(src, tgt, geom, edge_attr,
                      ep['A'], ep['B'], ep['b1'], ep['ar'], ep['ae'],
                      ep['g1'], ep['bg1'], ep['W2'], ep['b2'],
                      p['att_w'].reshape(1, _H), p['att_b'].reshape(1, 1))
        part = _scatter_add(ef, row, zeros_h)
        h = _tc_node(h, part,
                     p['n_w1'][:_H], p['n_w1'][_H:], p['n_b1'].reshape(1, _H),
                     p['n_ln_g'].reshape(1, _H), p['n_ln_b'].reshape(1, _H),
                     p['n_w2'], p['n_b2'].reshape(1, _H))

    src, tgt = _gather2(h, row, col)
    eq = params['eq']
    eqp = _edge_params(eq, w1_key='w1', b1_key='b1', g_key='ln1_g', bg_key='ln1_b',
                       w2_key='w2', b2_key='b2')
    trans = _tc_eq(src, tgt, geom, edge_attr,
                   eqp['A'], eqp['B'], eqp['b1'], eqp['ar'], eqp['ae'],
                   eqp['g1'], eqp['bg1'], eqp['W2'], eqp['b2'],
                   eq['ln2_g'].reshape(1, _H), eq['ln2_b'].reshape(1, _H),
                   eq['w3'].reshape(1, _H))
    partc = _scatter_add(trans, row, zeros_h)
    cnew = _tc_coord(cpad128, partc)
    return h, cnew[:, :3]
